# baseline (speedup 1.0000x reference)
import jax
import jax.numpy as jnp
from jax import lax
from jax.experimental import pallas as pl
from jax.experimental.pallas import tpu as pltpu
from jax.experimental.pallas import tpu_sc as plsc

B, C, H, W = 64, 3, 512, 512
PH, PW = 64, 64

_NC, _NS = 2, 16
_NW = _NC * _NS
_BPW = B // _NW
_L = 16
_WR, _WC = 72, 256
_NG = 5


_GATHER_DNUMS = lax.GatherDimensionNumbers(
    offset_dims=(), collapsed_slice_dims=(0,), start_index_map=(0,))


def _perm(v, idx):
  return lax.gather(v, idx[:, None], _GATHER_DNUMS, (1,),
                    mode=lax.GatherScatterMode.PROMISE_IN_BOUNDS)


def _sc_body(img_ref, wargs_ref, patch_ref, out_ref, patch_v, wargs_v, win_v):
  wid = lax.axis_index("s") * _NC + lax.axis_index("c")

  pltpu.sync_copy(patch_ref, patch_v)
  pltpu.sync_copy(wargs_ref.at[wid], wargs_v)
  vals = wargs_v[...]

  lanes = lax.iota(jnp.int32, _L)

  for j in range(_BPW):
    b = wid * _BPW + j
    pltpu.sync_copy(img_ref.at[b], out_ref.at[b])

    r = vals[2 * j]
    c = vals[2 * j + 1]
    r0 = pl.multiple_of((r // 8) * 8, 8)
    c0 = pl.multiple_of(jnp.minimum((c // 128) * 128, W - _WC), 128)
    dr = r - r0
    dc = c - c0
    g0 = pl.multiple_of((dc // _L) * _L, _L)

    for ch in range(C):
      pltpu.sync_copy(img_ref.at[b, ch, pl.ds(r0, _WR), pl.ds(c0, _WC)],
                      win_v)

      sh = dc - g0
      idx = (lanes - sh) % _L
      lo = lanes < sh

      @pl.loop(0, PH)
      def _(pi):
        wi = dr + pi
        segs = [patch_v[ch, pi, pl.ds(k * _L, _L)] for k in range(PW // _L)]
        perms = [_perm(v, idx) for v in segs]
        prev = [perms[0]] + perms
        cur_ = perms + [perms[-1]]
        for g in range(_NG):
          m0 = g0 + g * _L
          col = m0 + lanes - dc
          ok = (col >= 0) & (col < PW)
          pv = jnp.where(lo, prev[g], cur_[g])
          cur = win_v[wi, pl.ds(m0, _L)]
          win_v[wi, pl.ds(m0, _L)] = jnp.where(ok, pv, cur)

      pltpu.sync_copy(win_v,
                      out_ref.at[b, ch, pl.ds(r0, _WR), pl.ds(c0, _WC)])


def kernel(image, top_left_rows, top_left_cols, learned_patch):
  patch = learned_patch[0]

  rc = jnp.stack([top_left_rows, top_left_cols], axis=1).reshape(_NW, 2 * _BPW)
  wargs = jnp.zeros((_NW, _L), jnp.int32).at[:, : 2 * _BPW].set(rc)

  k = pl.kernel(
      _sc_body,
      out_type=jax.ShapeDtypeStruct((B, C, H, W), jnp.float32),
      mesh=plsc.VectorSubcoreMesh(core_axis_name="c", subcore_axis_name="s"),
      scratch_types=[
          pltpu.VMEM((C, PH, PW), jnp.float32),
          pltpu.VMEM((_L,), jnp.int32),
          pltpu.VMEM((_WR, _WC), jnp.float32),
      ],
  )
  return k(image, wargs, patch)


# speedup vs baseline: 35.7195x; 35.7195x over previous
import jax
import jax.numpy as jnp
from jax import lax
from jax.experimental import pallas as pl
from jax.experimental.pallas import tpu as pltpu
from jax.experimental.pallas import tpu_sc as plsc

B, C, H, W = 64, 3, 512, 512
PH, PW = 64, 64

_NC, _NS = 2, 16
_NW = _NC * _NS
_BPW = B // _NW
_L = 16
_WR, _WC = 72, 256
_NG = 5


_GATHER_DNUMS = lax.GatherDimensionNumbers(
    offset_dims=(), collapsed_slice_dims=(0,), start_index_map=(0,))


def _perm(v, idx):
  return lax.gather(v, idx[:, None], _GATHER_DNUMS, (1,),
                    mode=lax.GatherScatterMode.PROMISE_IN_BOUNDS)


def _sc_body(img_ref, wargs_ref, patch_ref, patch_v, wargs_v, win_v):
  wid = lax.axis_index("s") * _NC + lax.axis_index("c")

  pltpu.sync_copy(patch_ref, patch_v)
  pltpu.sync_copy(wargs_ref.at[wid], wargs_v)
  vals = wargs_v[...]

  lanes = lax.iota(jnp.int32, _L)

  for j in range(_BPW):
    b = wid * _BPW + j
    r = vals[2 * j]
    c = vals[2 * j + 1]
    r0 = pl.multiple_of((r // 8) * 8, 8)
    c0 = pl.multiple_of(jnp.minimum((c // 128) * 128, W - _WC), 128)
    dr = r - r0
    dc = c - c0
    g0 = pl.multiple_of((dc // _L) * _L, _L)

    for ch in range(C):
      pltpu.sync_copy(img_ref.at[b, ch, pl.ds(r0, _WR), pl.ds(c0, _WC)],
                      win_v)

      sh = dc - g0
      idx = (lanes - sh) % _L
      lo = lanes < sh

      @pl.loop(0, PH)
      def _(pi):
        wi = dr + pi
        segs = [patch_v[ch, pi, pl.ds(k * _L, _L)] for k in range(PW // _L)]
        perms = [_perm(v, idx) for v in segs]
        prev = [perms[0]] + perms
        cur_ = perms + [perms[-1]]
        for g in range(_NG):
          m0 = g0 + g * _L
          col = m0 + lanes - dc
          ok = (col >= 0) & (col < PW)
          pv = jnp.where(lo, prev[g], cur_[g])
          cur = win_v[wi, pl.ds(m0, _L)]
          win_v[wi, pl.ds(m0, _L)] = jnp.where(ok, pv, cur)

      pltpu.sync_copy(win_v,
                      img_ref.at[b, ch, pl.ds(r0, _WR), pl.ds(c0, _WC)])


def kernel(image, top_left_rows, top_left_cols, learned_patch):
  patch = learned_patch[0]

  rc = jnp.stack([top_left_rows, top_left_cols], axis=1).reshape(_NW, 2 * _BPW)
  wargs = jnp.zeros((_NW, _L), jnp.int32).at[:, : 2 * _BPW].set(rc)

  k = pl.kernel(
      _sc_body,
      out_type=(),
      mesh=plsc.VectorSubcoreMesh(core_axis_name="c", subcore_axis_name="s"),
      scratch_types=[
          pltpu.VMEM((C, PH, PW), jnp.float32),
          pltpu.VMEM((_L,), jnp.int32),
          pltpu.VMEM((_WR, _WC), jnp.float32),
      ],
  )
  img_ref = jax.new_ref(image)
  k(img_ref, wargs, patch)
  return img_ref[...]
